# baseline (speedup 1.0000x reference)
import functools

import jax
import jax.numpy as jnp
from jax.experimental import pallas as pl
from jax.experimental.pallas import tpu as pltpu

_PRE_TOP_K = 1000
_KEEP_TOP_K = 100
_IOU_THR = 0.65
_SCORE_THR = 0.25
_N = 20000
_NP = 20480
_ROWS = 160
_C = 80
_BN = 2000
_TOPP = 1024


def _dense_body(cls_ref, bx_ref, by_ref, bw_ref, bh_ref,
                p0_ref, p1_ref, p2_ref, p3_ref, st_ref,
                tab_ref, sc_ref):
    s = jax.nn.sigmoid(cls_ref[...])
    m = jnp.max(s, axis=1, keepdims=True)
    iota = jax.lax.broadcasted_iota(jnp.int32, s.shape, 1).astype(jnp.float32)
    lab = jnp.min(jnp.where(s == m, iota, jnp.float32(_C)), axis=1,
                  keepdims=True)

    stride = st_ref[...]
    xc = (p0_ref[...] + p2_ref[...]) * 0.5
    yc = (p1_ref[...] + p3_ref[...]) * 0.5
    w = p2_ref[...] - p0_ref[...]
    h = p3_ref[...] - p1_ref[...]
    sx = jax.nn.sigmoid(bx_ref[...])
    sy = jax.nn.sigmoid(by_ref[...])
    sw = jax.nn.sigmoid(bw_ref[...])
    sh = jax.nn.sigmoid(bh_ref[...])
    xcp = (sx - 0.5) * 2.0 * stride + xc
    ycp = (sy - 0.5) * 2.0 * stride + yc
    wp = (sw * 2.0) ** 2 * w
    hp = (sh * 2.0) ** 2 * h
    x1 = xcp - wp * 0.5
    y1 = ycp - hp * 0.5
    x2 = xcp + wp * 0.5
    y2 = ycp + hp * 0.5
    zero = jnp.zeros_like(m)
    tab_ref[...] = jnp.concatenate(
        [x1, y1, x2, y2, m, lab] + [zero] * 10, axis=1)
    sc_ref[...] = m


def _dense_stage(cls2d, bcols, pcols, stcol):
    grid = _N // _BN
    col_spec = pl.BlockSpec((_BN, 1), lambda i: (i, 0))
    return pl.pallas_call(
        _dense_body,
        grid=(grid,),
        in_specs=[pl.BlockSpec((_BN, _C), lambda i: (i, 0))] + [col_spec] * 9,
        out_specs=[pl.BlockSpec((_BN, 16), lambda i: (i, 0)), col_spec],
        out_shape=[jax.ShapeDtypeStruct((_N, 16), jnp.float32),
                   jax.ShapeDtypeStruct((_N, 1), jnp.float32)],
    )(cls2d, *bcols, *pcols, stcol)


def _select_body(s_ref, dest_ref):
    xb = jax.lax.bitcast_convert_type(s_ref[...], jnp.int32)

    def bs_body(_, lohi):
        lo, hi = lohi
        mid = (lo + hi) >> 1
        cnt = jnp.sum((xb >= mid).astype(jnp.int32))
        ge = cnt >= _PRE_TOP_K
        return (jnp.where(ge, mid, lo), jnp.where(ge, hi, mid))

    lo, hi = jax.lax.fori_loop(
        0, 30, bs_body, (jnp.int32(0), jnp.int32(0x40000000)))
    t = lo
    cnt_gt = jnp.sum((xb > t).astype(jnp.int32))
    m = (_PRE_TOP_K - cnt_gt).astype(jnp.float32)

    gt = (xb > t).astype(jnp.float32)
    eq = (xb == t).astype(jnp.float32)

    i0r = jax.lax.broadcasted_iota(jnp.int32, (_ROWS, _ROWS), 0)
    i1r = jax.lax.broadcasted_iota(jnp.int32, (_ROWS, _ROWS), 1)
    t160 = (i1r < i0r).astype(jnp.float32)
    i0c = jax.lax.broadcasted_iota(jnp.int32, (128, 128), 0)
    i1c = jax.lax.broadcasted_iota(jnp.int32, (128, 128), 1)
    ut128 = (i0c < i1c).astype(jnp.float32)
    ones = jnp.ones((128, 1), jnp.float32)

    def prefix(mk):
        offs = jnp.dot(jnp.dot(t160, mk, preferred_element_type=jnp.float32),
                       ones, preferred_element_type=jnp.float32)
        rowp = jnp.dot(mk, ut128, preferred_element_type=jnp.float32)
        return offs + rowp

    pgt = prefix(gt)
    peq = prefix(eq)
    keep = (gt > 0.0) | ((eq > 0.0) & (peq < m))
    pos = pgt + jnp.minimum(peq, m)
    dest_ref[...] = jnp.where(keep, pos.astype(jnp.int32), -1)


def _select_stage(s2d):
    return pl.pallas_call(
        _select_body,
        in_specs=[pl.BlockSpec((_ROWS, 128), lambda: (0, 0))],
        out_specs=pl.BlockSpec((_ROWS, 128), lambda: (0, 0)),
        out_shape=jax.ShapeDtypeStruct((_ROWS, 128), jnp.int32),
    )(s2d)


def _compact_stage(tabp, dest):
    iota = jnp.arange(_NP, dtype=jnp.int32)
    src = jnp.full((_TOPP,), _N, jnp.int32)
    src = src.at[jnp.where(dest >= 0, dest, _TOPP)].set(iota, mode="drop")
    tab1k = tabp[src]
    rows8 = tab1k[:, :8].T
    return tab1k, rows8


def _final_body(tab_ref, rows_ref, out_ref, stab_ref, m_ref):
    f32 = jnp.float32
    i0 = jax.lax.broadcasted_iota(jnp.int32, (_TOPP, _TOPP), 0)
    i1 = jax.lax.broadcasted_iota(jnp.int32, (_TOPP, _TOPP), 1)

    s_col = tab_ref[:, 4:5]
    s_row = rows_ref[4:5, :]
    g = ((s_col > s_row) | ((s_col == s_row) & (i0 < i1))).astype(f32)
    ones_r = jnp.ones((1, _TOPP), f32)
    ones_c = jnp.ones((_TOPP, 1), f32)
    rank_row = jnp.dot(ones_r, g, preferred_element_type=f32)
    beats = jnp.dot(g, ones_c, preferred_element_type=f32)
    rank_col = (_TOPP - 1.0) - beats

    p = (i0.astype(f32) == rank_row).astype(f32)
    pt = (rank_col == i1.astype(f32)).astype(f32)
    stab = jnp.dot(p, tab_ref[...], preferred_element_type=f32,
                   precision=jax.lax.Precision.HIGHEST)
    stab_ref[...] = stab
    srows = jnp.dot(rows_ref[...], pt, preferred_element_type=f32,
                    precision=jax.lax.Precision.HIGHEST)

    x1r, y1r, x2r, y2r = (stab_ref[:, 0:1], stab_ref[:, 1:2],
                          stab_ref[:, 2:3], stab_ref[:, 3:4])
    sr = stab_ref[:, 4:5]
    x1c, y1c, x2c, y2c = (srows[0:1, :], srows[1:2, :],
                          srows[2:3, :], srows[3:4, :])
    ix1 = jnp.maximum(x1r, x1c)
    iy1 = jnp.maximum(y1r, y1c)
    ix2 = jnp.minimum(x2r, x2c)
    iy2 = jnp.minimum(y2r, y2c)
    inter = jnp.clip(ix2 - ix1, 0.0) * jnp.clip(iy2 - iy1, 0.0)
    ar = (x2r - x1r) * (y2r - y1r)
    ac = (x2c - x1c) * (y2c - y1c)
    iou = inter / (ar + ac - inter + 1e-7)
    m_ref[...] = ((iou > _IOU_THR) & (i1 > i0)
                  & (sr > _SCORE_THR)).astype(f32)

    lane = jax.lax.broadcasted_iota(jnp.int32, (1, _TOPP), 1)

    def body(i, supp):
        mrow = m_ref[pl.ds(i, 1), :]
        onehot = (lane == i).astype(f32)
        si = jnp.sum(supp * onehot)
        return jnp.maximum(supp, mrow * (1.0 - si))

    supp = jax.lax.fori_loop(0, _PRE_TOP_K, body,
                             jnp.zeros((1, _TOPP), f32))

    scor_row = srows[4:5, :]
    real = lane < _PRE_TOP_K
    kept = (supp == 0.0) & (scor_row > _SCORE_THR) & real
    keptf = kept.astype(f32)
    nkf = jnp.where(real, 1.0 - keptf, 0.0)
    ut = (i0 < i1).astype(f32)
    pref_kept = jnp.dot(keptf, ut, preferred_element_type=f32)
    pref_nk = jnp.dot(nkf, ut, preferred_element_type=f32)
    ktot = jnp.sum(keptf)
    slot = jnp.where(kept, pref_kept, ktot + pref_nk)

    k0 = jax.lax.broadcasted_iota(jnp.int32, (128, _TOPP), 0).astype(f32)
    p2 = ((k0 == slot) & real).astype(f32)
    out = jnp.dot(p2, stab_ref[...], preferred_element_type=f32,
                  precision=jax.lax.Precision.HIGHEST)
    krow = (jax.lax.broadcasted_iota(jnp.int32, (128, 1), 0)
            .astype(f32) < ktot)
    out_ref[...] = out
    out_ref[:, 4:5] = jnp.where(krow, out[:, 4:5], -1.0)


def _final_stage(tab1k, rows8):
    return pl.pallas_call(
        _final_body,
        in_specs=[pl.BlockSpec((_TOPP, 16), lambda: (0, 0)),
                  pl.BlockSpec((8, _TOPP), lambda: (0, 0))],
        out_specs=pl.BlockSpec((128, 16), lambda: (0, 0)),
        out_shape=jax.ShapeDtypeStruct((128, 16), jnp.float32),
        scratch_shapes=[pltpu.VMEM((_TOPP, 16), jnp.float32),
                        pltpu.VMEM((_TOPP, _TOPP), jnp.float32)],
    )(tab1k, rows8)


def kernel(cls_scores, bbox_preds, priors, strides):
    cls2d = cls_scores.reshape(_N, _C)
    bcols = [bbox_preds[0, :, k:k + 1] for k in range(4)]
    pcols = [priors[:, k:k + 1] for k in range(4)]
    stcol = strides.reshape(_N, 1)

    tab, scol = _dense_stage(cls2d, bcols, pcols, stcol)
    tabp = jnp.concatenate(
        [tab, jnp.zeros((_NP - _N + 1, 16), jnp.float32)], axis=0)
    s2d = jnp.concatenate(
        [scol[:, 0], jnp.zeros((_NP - _N,), jnp.float32)]).reshape(_ROWS, 128)

    dest = _select_stage(s2d).reshape(_NP)
    tab1k, rows8 = _compact_stage(tabp, dest)
    out = _final_stage(tab1k, rows8)
    return out[:_KEEP_TOP_K, :6][None]


# speedup vs baseline: 1.1726x; 1.1726x over previous
import functools

import jax
import jax.numpy as jnp
from jax.experimental import pallas as pl
from jax.experimental.pallas import tpu as pltpu
from jax.experimental.pallas import tpu_sc as plsc

_PRE_TOP_K = 1000
_KEEP_TOP_K = 100
_IOU_THR = 0.65
_SCORE_THR = 0.25
_N = 20000
_NP = 20480
_ROWS = 160
_C = 80
_BN = 2000
_TOPP = 1024


def _dense_body(cls_ref, bx_ref, by_ref, bw_ref, bh_ref,
                p0_ref, p1_ref, p2_ref, p3_ref, st_ref,
                tab_ref, sc_ref):
    s = jax.nn.sigmoid(cls_ref[...])
    m = jnp.max(s, axis=1, keepdims=True)
    iota = jax.lax.broadcasted_iota(jnp.int32, s.shape, 1).astype(jnp.float32)
    lab = jnp.min(jnp.where(s == m, iota, jnp.float32(_C)), axis=1,
                  keepdims=True)

    stride = st_ref[...]
    xc = (p0_ref[...] + p2_ref[...]) * 0.5
    yc = (p1_ref[...] + p3_ref[...]) * 0.5
    w = p2_ref[...] - p0_ref[...]
    h = p3_ref[...] - p1_ref[...]
    sx = jax.nn.sigmoid(bx_ref[...])
    sy = jax.nn.sigmoid(by_ref[...])
    sw = jax.nn.sigmoid(bw_ref[...])
    sh = jax.nn.sigmoid(bh_ref[...])
    xcp = (sx - 0.5) * 2.0 * stride + xc
    ycp = (sy - 0.5) * 2.0 * stride + yc
    wp = (sw * 2.0) ** 2 * w
    hp = (sh * 2.0) ** 2 * h
    x1 = xcp - wp * 0.5
    y1 = ycp - hp * 0.5
    x2 = xcp + wp * 0.5
    y2 = ycp + hp * 0.5
    zero = jnp.zeros_like(m)
    tab_ref[...] = jnp.concatenate(
        [x1, y1, x2, y2, m, lab] + [zero] * 10, axis=1)
    sc_ref[...] = m


def _dense_stage(cls2d, bcols, pcols, stcol):
    grid = _N // _BN
    col_spec = pl.BlockSpec((_BN, 1), lambda i: (i, 0))
    return pl.pallas_call(
        _dense_body,
        grid=(grid,),
        in_specs=[pl.BlockSpec((_BN, _C), lambda i: (i, 0))] + [col_spec] * 9,
        out_specs=[pl.BlockSpec((_BN, 16), lambda i: (i, 0)), col_spec],
        out_shape=[jax.ShapeDtypeStruct((_N, 16), jnp.float32),
                   jax.ShapeDtypeStruct((_N, 1), jnp.float32)],
    )(cls2d, *bcols, *pcols, stcol)


def _select_body(s_ref, dest_ref):
    xb = jax.lax.bitcast_convert_type(s_ref[...], jnp.int32)

    def bs_body(_, lohi):
        lo, hi = lohi
        mid = (lo + hi) >> 1
        cnt = jnp.sum((xb >= mid).astype(jnp.int32))
        ge = cnt >= _PRE_TOP_K
        return (jnp.where(ge, mid, lo), jnp.where(ge, hi, mid))

    lo, hi = jax.lax.fori_loop(
        0, 30, bs_body, (jnp.int32(0), jnp.int32(0x40000000)))
    t = lo
    cnt_gt = jnp.sum((xb > t).astype(jnp.int32))
    m = (_PRE_TOP_K - cnt_gt).astype(jnp.float32)

    gt = (xb > t).astype(jnp.float32)
    eq = (xb == t).astype(jnp.float32)

    i0r = jax.lax.broadcasted_iota(jnp.int32, (_ROWS, _ROWS), 0)
    i1r = jax.lax.broadcasted_iota(jnp.int32, (_ROWS, _ROWS), 1)
    t160 = (i1r < i0r).astype(jnp.float32)
    i0c = jax.lax.broadcasted_iota(jnp.int32, (128, 128), 0)
    i1c = jax.lax.broadcasted_iota(jnp.int32, (128, 128), 1)
    ut128 = (i0c < i1c).astype(jnp.float32)
    ones = jnp.ones((128, 1), jnp.float32)

    def prefix(mk):
        offs = jnp.dot(jnp.dot(t160, mk, preferred_element_type=jnp.float32),
                       ones, preferred_element_type=jnp.float32)
        rowp = jnp.dot(mk, ut128, preferred_element_type=jnp.float32)
        return offs + rowp

    pgt = prefix(gt)
    peq = prefix(eq)
    keep = (gt > 0.0) | ((eq > 0.0) & (peq < m))
    pos = pgt + jnp.minimum(peq, m)
    dest_ref[...] = jnp.where(keep, pos.astype(jnp.int32), -1)


def _select_stage(s2d):
    return pl.pallas_call(
        _select_body,
        in_specs=[pl.BlockSpec((_ROWS, 128), lambda: (0, 0))],
        out_specs=pl.BlockSpec((_ROWS, 128), lambda: (0, 0)),
        out_shape=jax.ShapeDtypeStruct((_ROWS, 128), jnp.int32),
    )(s2d)


_SLOTS = _TOPP // 32


def _compact_body(dest_hbm, tabp_hbm, tab1k_hbm, rows8_hbm,
                  dest_v, srcbuf, rows_v, fld_v, sem):
    wid = jax.lax.axis_index("s") * 2 + jax.lax.axis_index("c")
    base = wid * _SLOTS
    pltpu.sync_copy(dest_hbm, dest_v)
    for q in range(_SLOTS // 16):
        srcbuf[pl.ds(q * 16, 16)] = jnp.full((16,), _N, jnp.int32)
    lane = jax.lax.iota(jnp.int32, 16)

    def scan(k, _):
        d = dest_v[pl.ds(k * 16, 16)]
        m = (d >= base) & (d < base + _SLOTS)
        idx = jnp.clip(d - base, 0, _SLOTS - 1)
        plsc.store_scatter(srcbuf, [idx], lane + k * 16, mask=m)
        return 0

    jax.lax.fori_loop(0, _NP // 16, scan, 0)
    pltpu.async_copy(tabp_hbm.at[srcbuf], rows_v, sem).wait()
    pltpu.sync_copy(rows_v, tab1k_hbm.at[pl.ds(base, _SLOTS)])
    for f in range(8):
        a = plsc.load_gather(rows_v, [jax.lax.iota(jnp.int32, 16),
                                      jnp.full((16,), f, jnp.int32)])
        b = plsc.load_gather(rows_v, [jax.lax.iota(jnp.int32, 16) + 16,
                                      jnp.full((16,), f, jnp.int32)])
        fld_v[pl.ds(0, 16)] = a
        fld_v[pl.ds(16, 16)] = b
        pltpu.sync_copy(fld_v, rows8_hbm.at[f, pl.ds(base, _SLOTS)])


def _compact_stage(tabp, dest):
    mesh = plsc.VectorSubcoreMesh(core_axis_name="c", subcore_axis_name="s")
    run = pl.kernel(
        _compact_body,
        mesh=mesh,
        out_type=[jax.ShapeDtypeStruct((_TOPP, 16), jnp.float32),
                  jax.ShapeDtypeStruct((8, _TOPP), jnp.float32)],
        scratch_types=[pltpu.VMEM((_NP,), jnp.int32),
                       pltpu.VMEM((_SLOTS,), jnp.int32),
                       pltpu.VMEM((_SLOTS, 16), jnp.float32),
                       pltpu.VMEM((_SLOTS,), jnp.float32),
                       pltpu.SemaphoreType.DMA],
        compiler_params=pltpu.CompilerParams(needs_layout_passes=False, use_tc_tiling_on_sc=False),
    )
    return run(dest, tabp)


def _final_body(tab_ref, rows_ref, out_ref, stab_ref, m_ref):
    f32 = jnp.float32
    i0 = jax.lax.broadcasted_iota(jnp.int32, (_TOPP, _TOPP), 0)
    i1 = jax.lax.broadcasted_iota(jnp.int32, (_TOPP, _TOPP), 1)

    s_col = tab_ref[:, 4:5]
    s_row = rows_ref[4:5, :]
    g = ((s_col > s_row) | ((s_col == s_row) & (i0 < i1))).astype(f32)
    ones_r = jnp.ones((1, _TOPP), f32)
    ones_c = jnp.ones((_TOPP, 1), f32)
    rank_row = jnp.dot(ones_r, g, preferred_element_type=f32)
    beats = jnp.dot(g, ones_c, preferred_element_type=f32)
    rank_col = (_TOPP - 1.0) - beats

    p = (i0.astype(f32) == rank_row).astype(f32)
    pt = (rank_col == i1.astype(f32)).astype(f32)
    stab = jnp.dot(p, tab_ref[...], preferred_element_type=f32,
                   precision=jax.lax.Precision.HIGHEST)
    stab_ref[...] = stab
    srows = jnp.dot(rows_ref[...], pt, preferred_element_type=f32,
                    precision=jax.lax.Precision.HIGHEST)

    x1r, y1r, x2r, y2r = (stab_ref[:, 0:1], stab_ref[:, 1:2],
                          stab_ref[:, 2:3], stab_ref[:, 3:4])
    sr = stab_ref[:, 4:5]
    x1c, y1c, x2c, y2c = (srows[0:1, :], srows[1:2, :],
                          srows[2:3, :], srows[3:4, :])
    ix1 = jnp.maximum(x1r, x1c)
    iy1 = jnp.maximum(y1r, y1c)
    ix2 = jnp.minimum(x2r, x2c)
    iy2 = jnp.minimum(y2r, y2c)
    inter = jnp.clip(ix2 - ix1, 0.0) * jnp.clip(iy2 - iy1, 0.0)
    ar = (x2r - x1r) * (y2r - y1r)
    ac = (x2c - x1c) * (y2c - y1c)
    iou = inter / (ar + ac - inter + 1e-7)
    m_ref[...] = ((iou > _IOU_THR) & (i1 > i0)
                  & (sr > _SCORE_THR)).astype(f32)

    lane = jax.lax.broadcasted_iota(jnp.int32, (1, _TOPP), 1)

    def body(i, supp):
        mrow = m_ref[pl.ds(i, 1), :]
        onehot = (lane == i).astype(f32)
        si = jnp.sum(supp * onehot)
        return jnp.maximum(supp, mrow * (1.0 - si))

    supp = jax.lax.fori_loop(0, _PRE_TOP_K, body,
                             jnp.zeros((1, _TOPP), f32))

    scor_row = srows[4:5, :]
    real = lane < _PRE_TOP_K
    kept = (supp == 0.0) & (scor_row > _SCORE_THR) & real
    keptf = kept.astype(f32)
    nkf = jnp.where(real, 1.0 - keptf, 0.0)
    ut = (i0 < i1).astype(f32)
    pref_kept = jnp.dot(keptf, ut, preferred_element_type=f32)
    pref_nk = jnp.dot(nkf, ut, preferred_element_type=f32)
    ktot = jnp.sum(keptf)
    slot = jnp.where(kept, pref_kept, ktot + pref_nk)

    k0 = jax.lax.broadcasted_iota(jnp.int32, (128, _TOPP), 0).astype(f32)
    p2 = ((k0 == slot) & real).astype(f32)
    out = jnp.dot(p2, stab_ref[...], preferred_element_type=f32,
                  precision=jax.lax.Precision.HIGHEST)
    krow = (jax.lax.broadcasted_iota(jnp.int32, (128, 1), 0)
            .astype(f32) < ktot)
    out_ref[...] = out
    out_ref[:, 4:5] = jnp.where(krow, out[:, 4:5], -1.0)


def _final_stage(tab1k, rows8):
    return pl.pallas_call(
        _final_body,
        in_specs=[pl.BlockSpec((_TOPP, 16), lambda: (0, 0)),
                  pl.BlockSpec((8, _TOPP), lambda: (0, 0))],
        out_specs=pl.BlockSpec((128, 16), lambda: (0, 0)),
        out_shape=jax.ShapeDtypeStruct((128, 16), jnp.float32),
        scratch_shapes=[pltpu.VMEM((_TOPP, 16), jnp.float32),
                        pltpu.VMEM((_TOPP, _TOPP), jnp.float32)],
    )(tab1k, rows8)


def kernel(cls_scores, bbox_preds, priors, strides):
    cls2d = cls_scores.reshape(_N, _C)
    bcols = [bbox_preds[0, :, k:k + 1] for k in range(4)]
    pcols = [priors[:, k:k + 1] for k in range(4)]
    stcol = strides.reshape(_N, 1)

    tab, scol = _dense_stage(cls2d, bcols, pcols, stcol)
    tabp = jnp.concatenate(
        [tab, jnp.zeros((_NP - _N + 1, 16), jnp.float32)], axis=0)
    s2d = jnp.concatenate(
        [scol[:, 0], jnp.zeros((_NP - _N,), jnp.float32)]).reshape(_ROWS, 128)

    dest = _select_stage(s2d).reshape(_NP)
    tab1k, rows8 = _compact_stage(tabp, dest)
    out = _final_stage(tab1k, rows8)
    return out[:_KEEP_TOP_K, :6][None]


# speedup vs baseline: 1.4127x; 1.2047x over previous
import functools

import jax
import jax.numpy as jnp
from jax.experimental import pallas as pl
from jax.experimental.pallas import tpu as pltpu
from jax.experimental.pallas import tpu_sc as plsc

_PRE_TOP_K = 1000
_KEEP_TOP_K = 100
_IOU_THR = 0.65
_SCORE_THR = 0.25
_N = 20000
_NP = 20480
_ROWS = 160
_C = 80
_BN = 2000
_TOPP = 1024


def _dense_body(cls_ref, bb_ref, pr_ref, st_ref, tab_ref, sc_ref):
    s = jax.nn.sigmoid(cls_ref[...])
    m = jnp.max(s, axis=1, keepdims=True)
    iota = jax.lax.broadcasted_iota(jnp.int32, s.shape, 1).astype(jnp.float32)
    lab = jnp.min(jnp.where(s == m, iota, jnp.float32(_C)), axis=1,
                  keepdims=True)

    stride = st_ref[...]
    p0, p1, p2, p3 = (pr_ref[:, 0:1], pr_ref[:, 1:2],
                      pr_ref[:, 2:3], pr_ref[:, 3:4])
    xc = (p0 + p2) * 0.5
    yc = (p1 + p3) * 0.5
    w = p2 - p0
    h = p3 - p1
    bb = jax.nn.sigmoid(bb_ref[...])
    sx, sy, sw, sh = bb[:, 0:1], bb[:, 1:2], bb[:, 2:3], bb[:, 3:4]
    xcp = (sx - 0.5) * 2.0 * stride + xc
    ycp = (sy - 0.5) * 2.0 * stride + yc
    wp = (sw * 2.0) ** 2 * w
    hp = (sh * 2.0) ** 2 * h
    x1 = xcp - wp * 0.5
    y1 = ycp - hp * 0.5
    x2 = xcp + wp * 0.5
    y2 = ycp + hp * 0.5
    zero = jnp.zeros_like(m)
    tab_ref[...] = jnp.concatenate(
        [x1, y1, x2, y2, m, lab] + [zero] * 10, axis=1)
    sc_ref[...] = m


def _dense_stage(cls2d, bb4, pr4, stcol):
    grid = _N // _BN
    return pl.pallas_call(
        _dense_body,
        grid=(grid,),
        in_specs=[pl.BlockSpec((_BN, _C), lambda i: (i, 0)),
                  pl.BlockSpec((_BN, 4), lambda i: (i, 0)),
                  pl.BlockSpec((_BN, 4), lambda i: (i, 0)),
                  pl.BlockSpec((_BN, 1), lambda i: (i, 0))],
        out_specs=[pl.BlockSpec((_BN, 16), lambda i: (i, 0)),
                   pl.BlockSpec((_BN, 1), lambda i: (i, 0))],
        out_shape=[jax.ShapeDtypeStruct((_N, 16), jnp.float32),
                   jax.ShapeDtypeStruct((_N, 1), jnp.float32)],
    )(cls2d, bb4, pr4, stcol)


def _select_body(s_ref, dest_ref):
    xb = jax.lax.bitcast_convert_type(s_ref[...], jnp.int32)

    def bs_body(_, lohi):
        lo, hi = lohi
        mid = (lo + hi) >> 1
        cnt = jnp.sum((xb >= mid).astype(jnp.int32))
        ge = cnt >= _PRE_TOP_K
        return (jnp.where(ge, mid, lo), jnp.where(ge, hi, mid))

    lo, hi = jax.lax.fori_loop(
        0, 30, bs_body, (jnp.int32(0), jnp.int32(0x40000000)))
    t = lo
    cnt_gt = jnp.sum((xb > t).astype(jnp.int32))
    m = (_PRE_TOP_K - cnt_gt).astype(jnp.float32)

    gt = (xb > t).astype(jnp.float32)
    eq = (xb == t).astype(jnp.float32)

    i0r = jax.lax.broadcasted_iota(jnp.int32, (_ROWS, _ROWS), 0)
    i1r = jax.lax.broadcasted_iota(jnp.int32, (_ROWS, _ROWS), 1)
    t160 = (i1r < i0r).astype(jnp.float32)
    i0c = jax.lax.broadcasted_iota(jnp.int32, (128, 128), 0)
    i1c = jax.lax.broadcasted_iota(jnp.int32, (128, 128), 1)
    ut128 = (i0c < i1c).astype(jnp.float32)
    ones = jnp.ones((128, 1), jnp.float32)

    def prefix(mk):
        offs = jnp.dot(jnp.dot(t160, mk, preferred_element_type=jnp.float32),
                       ones, preferred_element_type=jnp.float32)
        rowp = jnp.dot(mk, ut128, preferred_element_type=jnp.float32)
        return offs + rowp

    pgt = prefix(gt)
    peq = prefix(eq)
    keep = (gt > 0.0) | ((eq > 0.0) & (peq < m))
    pos = pgt + jnp.minimum(peq, m)
    dest_ref[...] = jnp.where(keep, pos.astype(jnp.int32), -1)


def _select_stage(s2d):
    return pl.pallas_call(
        _select_body,
        in_specs=[pl.BlockSpec((_ROWS, 128), lambda: (0, 0))],
        out_specs=pl.BlockSpec((_ROWS, 128), lambda: (0, 0)),
        out_shape=jax.ShapeDtypeStruct((_ROWS, 128), jnp.int32),
    )(s2d)


_SLOTS = _TOPP // 32


def _compact_body(dest_hbm, tabp_hbm, tab1k_hbm, rows8_hbm,
                  dest_v, srcbuf, rows_v, fld_v, sem):
    wid = jax.lax.axis_index("s") * 2 + jax.lax.axis_index("c")
    base = wid * _SLOTS
    pltpu.sync_copy(dest_hbm, dest_v)
    for q in range(_SLOTS // 16):
        srcbuf[pl.ds(q * 16, 16)] = jnp.zeros((16,), jnp.int32)
    lane = jax.lax.iota(jnp.int32, 16)

    def scan(k, _):
        d = dest_v[pl.ds(k * 16, 16)]
        m = (d >= base) & (d < base + _SLOTS)
        idx = jnp.clip(d - base, 0, _SLOTS - 1)
        plsc.store_scatter(srcbuf, [idx], lane + k * 16, mask=m)
        return 0

    jax.lax.fori_loop(0, _NP // 16, scan, 0)
    pltpu.async_copy(tabp_hbm.at[srcbuf], rows_v, sem).wait()
    pltpu.sync_copy(rows_v, tab1k_hbm.at[pl.ds(base, _SLOTS)])
    for f in range(8):
        a = plsc.load_gather(rows_v, [jax.lax.iota(jnp.int32, 16),
                                      jnp.full((16,), f, jnp.int32)])
        b = plsc.load_gather(rows_v, [jax.lax.iota(jnp.int32, 16) + 16,
                                      jnp.full((16,), f, jnp.int32)])
        fld_v[pl.ds(0, 16)] = a
        fld_v[pl.ds(16, 16)] = b
        pltpu.sync_copy(fld_v, rows8_hbm.at[f, pl.ds(base, _SLOTS)])


def _compact_stage(tabp, dest):
    mesh = plsc.VectorSubcoreMesh(core_axis_name="c", subcore_axis_name="s")
    run = pl.kernel(
        _compact_body,
        mesh=mesh,
        out_type=[jax.ShapeDtypeStruct((_TOPP, 16), jnp.float32),
                  jax.ShapeDtypeStruct((8, _TOPP), jnp.float32)],
        scratch_types=[pltpu.VMEM((_NP,), jnp.int32),
                       pltpu.VMEM((_SLOTS,), jnp.int32),
                       pltpu.VMEM((_SLOTS, 16), jnp.float32),
                       pltpu.VMEM((_SLOTS,), jnp.float32),
                       pltpu.SemaphoreType.DMA],
        compiler_params=pltpu.CompilerParams(needs_layout_passes=False, use_tc_tiling_on_sc=False),
    )
    return run(dest, tabp)


def _final_body(tab_ref, rows_ref, out_ref, stab_ref, m_ref):
    f32 = jnp.float32
    i0 = jax.lax.broadcasted_iota(jnp.int32, (_TOPP, _TOPP), 0)
    i1 = jax.lax.broadcasted_iota(jnp.int32, (_TOPP, _TOPP), 1)

    s_col = tab_ref[:, 4:5]
    s_row = rows_ref[4:5, :]
    realj = i0 < _PRE_TOP_K
    reali = i1 < _PRE_TOP_K
    cmp = (s_col > s_row) | ((s_col == s_row) & (i0 < i1))
    g = ((realj & reali & cmp) | (realj & ~reali)
         | (~realj & ~reali & (i0 < i1))).astype(f32)
    ones_r = jnp.ones((1, _TOPP), f32)
    ones_c = jnp.ones((_TOPP, 1), f32)
    rank_row = jnp.dot(ones_r, g, preferred_element_type=f32)
    beats = jnp.dot(g, ones_c, preferred_element_type=f32)
    rank_col = (_TOPP - 1.0) - beats

    p = (i0.astype(f32) == rank_row).astype(f32)
    pt = (rank_col == i1.astype(f32)).astype(f32)
    stab = jnp.dot(p, tab_ref[...], preferred_element_type=f32,
                   precision=jax.lax.Precision.HIGHEST)
    stab_ref[...] = stab
    srows = jnp.dot(rows_ref[...], pt, preferred_element_type=f32,
                    precision=jax.lax.Precision.HIGHEST)

    x1r, y1r, x2r, y2r = (stab_ref[:, 0:1], stab_ref[:, 1:2],
                          stab_ref[:, 2:3], stab_ref[:, 3:4])
    sr = stab_ref[:, 4:5]
    x1c, y1c, x2c, y2c = (srows[0:1, :], srows[1:2, :],
                          srows[2:3, :], srows[3:4, :])
    ix1 = jnp.maximum(x1r, x1c)
    iy1 = jnp.maximum(y1r, y1c)
    ix2 = jnp.minimum(x2r, x2c)
    iy2 = jnp.minimum(y2r, y2c)
    inter = jnp.clip(ix2 - ix1, 0.0) * jnp.clip(iy2 - iy1, 0.0)
    ar = (x2r - x1r) * (y2r - y1r)
    ac = (x2c - x1c) * (y2c - y1c)
    iou = inter / (ar + ac - inter + 1e-7)
    m_ref[...] = ((iou > _IOU_THR) & (i1 > i0) & (i0 < _PRE_TOP_K)
                  & (sr > _SCORE_THR)).astype(f32)

    lane128 = jax.lax.broadcasted_iota(jnp.int32, (1, 128), 1)
    supp = jnp.zeros((1, _TOPP), f32)
    for blk in range(_TOPP // 128):
        cs = blk * 128

        def inner(i, sblk, cs=cs):
            mrow = m_ref[pl.ds(cs + i, 1), :][:, cs:cs + 128]
            onehot = (lane128 == i).astype(f32)
            si = jnp.sum(sblk * onehot)
            return jnp.maximum(sblk, mrow * (1.0 - si))

        sblk = jax.lax.fori_loop(0, 128, inner, supp[:, cs:cs + 128])
        keepv = (sblk == 0.0).astype(f32)
        contrib = jnp.dot(keepv, m_ref[pl.ds(cs, 128), :],
                          preferred_element_type=f32)
        parts = ([supp[:, :cs]] if cs else []) + [sblk] + (
            [supp[:, cs + 128:]] if cs + 128 < _TOPP else [])
        merged = jnp.concatenate(parts, axis=1) if len(parts) > 1 else sblk
        supp = (merged + contrib > 0.0).astype(f32)

    lane = jax.lax.broadcasted_iota(jnp.int32, (1, _TOPP), 1)
    scor_row = srows[4:5, :]
    real = lane < _PRE_TOP_K
    kept = (supp == 0.0) & (scor_row > _SCORE_THR) & real
    keptf = kept.astype(f32)
    nkf = jnp.where(real, 1.0 - keptf, 0.0)
    ut = (i0 < i1).astype(f32)
    pref_kept = jnp.dot(keptf, ut, preferred_element_type=f32)
    pref_nk = jnp.dot(nkf, ut, preferred_element_type=f32)
    ktot = jnp.sum(keptf)
    slot = jnp.where(kept, pref_kept, ktot + pref_nk)

    k0 = jax.lax.broadcasted_iota(jnp.int32, (128, _TOPP), 0).astype(f32)
    p2 = ((k0 == slot) & real).astype(f32)
    out = jnp.dot(p2, stab_ref[...], preferred_element_type=f32,
                  precision=jax.lax.Precision.HIGHEST)
    krow = (jax.lax.broadcasted_iota(jnp.int32, (128, 1), 0)
            .astype(f32) < ktot)
    out_ref[...] = out
    out_ref[:, 4:5] = jnp.where(krow, out[:, 4:5], -1.0)


def _final_stage(tab1k, rows8):
    return pl.pallas_call(
        _final_body,
        in_specs=[pl.BlockSpec((_TOPP, 16), lambda: (0, 0)),
                  pl.BlockSpec((8, _TOPP), lambda: (0, 0))],
        out_specs=pl.BlockSpec((128, 16), lambda: (0, 0)),
        out_shape=jax.ShapeDtypeStruct((128, 16), jnp.float32),
        scratch_shapes=[pltpu.VMEM((_TOPP, 16), jnp.float32),
                        pltpu.VMEM((_TOPP, _TOPP), jnp.float32)],
    )(tab1k, rows8)


def kernel(cls_scores, bbox_preds, priors, strides):
    cls2d = cls_scores.reshape(_N, _C)
    stcol = strides.reshape(_N, 1)

    tab, scol = _dense_stage(cls2d, bbox_preds[0], priors, stcol)
    s2d = jnp.concatenate(
        [scol[:, 0], jnp.zeros((_NP - _N,), jnp.float32)]).reshape(_ROWS, 128)

    dest = _select_stage(s2d).reshape(_NP)
    tab1k, rows8 = _compact_stage(tab, dest)
    out = _final_stage(tab1k, rows8)
    return out[:_KEEP_TOP_K, :6][None]
